# baseline (speedup 1.0000x reference)
import functools

import jax
import jax.numpy as jnp
from jax import lax
from jax.experimental import pallas as pl
from jax.experimental.pallas import tpu as pltpu
from jax.experimental.pallas import tpu_sc as plsc

B = 4096
CTX = 20
D = 64
V = 100000

NC = 2
NS = 16
NW = NC * NS
B_PER_W = B // NW
CHUNK = 64
N_CHUNKS = B_PER_W // CHUNK

_sc_mesh = plsc.VectorSubcoreMesh(core_axis_name="c", subcore_axis_name="s")


@functools.partial(
    pl.kernel,
    out_type=jax.ShapeDtypeStruct((B, D), jnp.float32),
    mesh=_sc_mesh,
    scratch_types=[
        pltpu.VMEM((CHUNK * CTX,), jnp.int32),
        pltpu.VMEM((CHUNK * CTX, D), jnp.float32),
        pltpu.VMEM((B_PER_W, D), jnp.float32),
        pltpu.SemaphoreType.DMA,
    ],
    compiler_params=pltpu.CompilerParams(use_tc_tiling_on_sc=False),
)
def _pool_sc(idx_hbm, table_hbm, out_hbm, idx_v, rows_v, out_v, sem):
    wid = lax.axis_index("s") * NC + lax.axis_index("c")
    base = wid * B_PER_W
    for c in range(N_CHUNKS):
        pltpu.sync_copy(
            idx_hbm.at[pl.ds((base + c * CHUNK) * CTX, CHUNK * CTX)], idx_v)
        pltpu.async_copy(table_hbm.at[idx_v], rows_v, sem).wait()

        def row_body(b, _, c=c):
            for j in range(D // 16):
                sl = pl.ds(j * 16, 16)
                acc = rows_v[b * CTX, sl]
                for l in range(1, CTX):
                    acc = acc + rows_v[b * CTX + l, sl]
                out_v[c * CHUNK + b, sl] = acc * (1.0 / CTX)
            return 0

        lax.fori_loop(0, CHUNK, row_body, 0)
    pltpu.sync_copy(out_v, out_hbm.at[pl.ds(base, B_PER_W)])


VB = 1024


def _mm_body(x_ref, w_ref, b_ref, o_ref):
    o_ref[...] = (
        jnp.dot(x_ref[...], w_ref[...], preferred_element_type=jnp.float32)
        + b_ref[...]
    )


_matmul = pl.pallas_call(
    _mm_body,
    grid=(pl.cdiv(V, VB),),
    in_specs=[
        pl.BlockSpec((B, D), lambda j: (0, 0)),
        pl.BlockSpec((D, VB), lambda j: (0, j)),
        pl.BlockSpec((1, VB), lambda j: (0, j)),
    ],
    out_specs=pl.BlockSpec((B, VB), lambda j: (0, j)),
    out_shape=jax.ShapeDtypeStruct((B, V), jnp.float32),
    compiler_params=pltpu.CompilerParams(
        dimension_semantics=("arbitrary",)),
)


def kernel(inputs, embedding_table, fc_w, fc_b):
    idx = inputs.reshape(-1).astype(jnp.int32)
    pooled = _pool_sc(idx, embedding_table)
    return _matmul(pooled, fc_w, fc_b.reshape(1, V))
